# baseline (speedup 1.0000x reference)
import functools

import jax
import jax.numpy as jnp
from jax import lax
from jax.experimental import pallas as pl
from jax.experimental.pallas import tpu as pltpu
from jax.experimental.pallas import tpu_sc as plsc

VOCAB = 100000
EMBED = 64
B = 4096
L = 50

_NC = 2
_NS = 16
_NW = _NC * _NS
_CH = 128
_PADW = _CH + 1

_mesh = plsc.VectorSubcoreMesh(core_axis_name="c", subcore_axis_name="s")


@functools.partial(
    pl.kernel,
    mesh=_mesh,
    out_type=jax.ShapeDtypeStruct((L, 8, B // _CH, 8, _CH), jnp.float32),
    scratch_types=[
        pltpu.VMEM((_CH * L,), jnp.int32),
        pltpu.VMEM((EMBED, _PADW), jnp.int32),
        pltpu.VMEM((2, _CH, EMBED), jnp.float32),
        pltpu.VMEM((2, EMBED, _PADW), jnp.float32),
        pltpu.SemaphoreType.DMA,
        pltpu.SemaphoreType.DMA,
        pltpu.SemaphoreType.DMA,
    ],
    compiler_params=pltpu.CompilerParams(
        use_tc_tiling_on_sc=False, needs_layout_passes=False
    ),
)
def _gather(x_hbm, table_hbm, out_hbm, xbuf, xt, rows_v, obuf, gsem0, gsem1, osem):
    wid = lax.axis_index("s") * _NC + lax.axis_index("c")
    lane = lax.iota(jnp.int32, 16)
    rvec = [fc * 16 + lane for fc in range(EMBED // 16)]
    gsems = (gsem0, gsem1)

    pltpu.sync_copy(x_hbm.at[pl.ds(wid * _CH * L, _CH * L)], xbuf)
    xoffs = [0, 16, 32, L - 16]
    xvecs = [off + lane for off in xoffs]
    for bl in range(_CH):
        cbl = lane * 0 + bl
        vs = [xbuf[pl.ds(bl * L + off, 16)] for off in xoffs]
        for k in range(4):
            plsc.store_scatter(xt, [xvecs[k], cbl], vs[k])

    def gfire(i, sb):
        pltpu.async_copy(
            table_hbm.at[xt.at[i, pl.ds(0, _CH)]], rows_v.at[sb], gsems[sb]
        )

    def gwait(sb):
        pltpu.make_async_copy(
            table_hbm.at[pl.ds(0, _CH)], rows_v.at[0], gsems[sb]
        ).wait()

    def owait():
        for _ in range(8):
            pltpu.make_async_copy(
                obuf.at[0, pl.ds(0, 8), pl.ds(0, _CH)], out_hbm.at[0, 0, 0], osem
            ).wait()

    def transpose(sb):
        def bl_body(b16, _):
            for blo in range(8):
                bl = b16 * 8 + blo
                cbl = lane * 0 + bl
                vs = [
                    rows_v[sb, bl, pl.ds(fc * 16, 16)]
                    for fc in range(EMBED // 16)
                ]
                for fc in range(EMBED // 16):
                    plsc.store_scatter(obuf.at[sb], [rvec[fc], cbl], vs[fc])
            return 0

        lax.fori_loop(0, _CH // 8, bl_body, 0)

    gfire(0, 0)

    def pair(p, _):
        for sb in (0, 1):
            i = p * 2 + sb

            @pl.when(i + 1 < L)
            def _():
                gfire(i + 1, 1 - sb)

            gwait(sb)

            @pl.when(i >= 2)
            def _():
                owait()

            transpose(sb)
            for eh in range(8):
                pltpu.async_copy(
                    obuf.at[sb, pl.ds(eh * 8, 8), pl.ds(0, _CH)],
                    out_hbm.at[i, eh, wid],
                    osem,
                )
        return 0

    lax.fori_loop(0, L // 2, pair, 0)
    owait()
    owait()


def kernel(x, seq_lengths, table):
    del seq_lengths
    out5 = _gather(x.reshape(-1).astype(jnp.int32), table)
    return out5.transpose(2, 4, 0, 1, 3).reshape(B, L, EMBED)


# speedup vs baseline: 1.0178x; 1.0178x over previous
import functools

import jax
import jax.numpy as jnp
from jax import lax
from jax.experimental import pallas as pl
from jax.experimental.pallas import tpu as pltpu
from jax.experimental.pallas import tpu_sc as plsc

VOCAB = 100000
EMBED = 64
B = 4096
L = 50

_NC = 2
_NS = 16
_NW = _NC * _NS
_CH = 128
_PADW = _CH + 1

_mesh = plsc.VectorSubcoreMesh(core_axis_name="c", subcore_axis_name="s")


@functools.partial(
    pl.kernel,
    mesh=_mesh,
    out_type=jax.ShapeDtypeStruct((L, 8, B // _CH, 8, _CH), jnp.float32),
    scratch_types=[
        pltpu.VMEM((7, 8, _CH), jnp.int32),
        pltpu.VMEM((2, _CH, EMBED), jnp.float32),
        pltpu.VMEM((2, EMBED, _PADW), jnp.float32),
        pltpu.SemaphoreType.DMA,
        pltpu.SemaphoreType.DMA,
        pltpu.SemaphoreType.DMA,
    ],
    compiler_params=pltpu.CompilerParams(
        use_tc_tiling_on_sc=False, needs_layout_passes=False
    ),
)
def _gather(x_hbm, table_hbm, out_hbm, xbuf, rows_v, obuf, gsem0, gsem1, osem):
    wid = lax.axis_index("s") * _NC + lax.axis_index("c")
    lane = lax.iota(jnp.int32, 16)
    rvec = [fc * 16 + lane for fc in range(EMBED // 16)]
    gsems = (gsem0, gsem1)

    pltpu.sync_copy(x_hbm.at[:, wid], xbuf)

    def gfire(i, sb):
        pltpu.async_copy(
            table_hbm.at[xbuf.at[i // 8, i % 8]], rows_v.at[sb], gsems[sb]
        )

    def gwait(sb):
        pltpu.make_async_copy(
            table_hbm.at[pl.ds(0, _CH)], rows_v.at[0], gsems[sb]
        ).wait()

    def owait():
        for _ in range(8):
            pltpu.make_async_copy(
                obuf.at[0, pl.ds(0, 8), pl.ds(0, _CH)], out_hbm.at[0, 0, 0], osem
            ).wait()

    def transpose(sb):
        def bl_body(b16, _):
            for blo in range(8):
                bl = b16 * 8 + blo
                cbl = lane * 0 + bl
                vs = [
                    rows_v[sb, bl, pl.ds(fc * 16, 16)]
                    for fc in range(EMBED // 16)
                ]
                for fc in range(EMBED // 16):
                    plsc.store_scatter(obuf.at[sb], [rvec[fc], cbl], vs[fc])
            return 0

        lax.fori_loop(0, _CH // 8, bl_body, 0)

    gfire(0, 0)

    def pair(p, _):
        for sb in (0, 1):
            i = p * 2 + sb

            @pl.when(i + 1 < L)
            def _():
                gfire(i + 1, 1 - sb)

            gwait(sb)

            @pl.when(i >= 2)
            def _():
                owait()

            transpose(sb)
            for eh in range(8):
                pltpu.async_copy(
                    obuf.at[sb, pl.ds(eh * 8, 8), pl.ds(0, _CH)],
                    out_hbm.at[i, eh, wid],
                    osem,
                )
        return 0

    lax.fori_loop(0, L // 2, pair, 0)
    owait()
    owait()


def kernel(x, seq_lengths, table):
    del seq_lengths
    x7 = jnp.pad(x.T, ((0, 6), (0, 0))).reshape(7, 8, B // _CH, _CH)
    x7 = x7.transpose(0, 2, 1, 3)
    out5 = _gather(x7.astype(jnp.int32), table)
    return out5.transpose(2, 4, 0, 1, 3).reshape(B, L, EMBED)
